# baseline (speedup 1.0000x reference)
import jax
import jax.numpy as jnp
from jax import lax
from jax.experimental import pallas as pl
from jax.experimental.pallas import tpu as pltpu
from jax.experimental.pallas import tpu_sc as plsc

B = 16384
D = 32
L = 16
PACK = 128 // D

_info = plsc.get_sparse_core_info()
NC, NS = _info.num_cores, _info.num_subcores
NW = NC * NS
BPW = B // NW
CHUNK = 128
NCHUNK = BPW // CHUNK
BLKS = CHUNK // L


def _sc_body(uidx_hbm, iidx_hbm, utab_hbm, itab_hbm, out_hbm,
             uidx_v, iidx_v, urow_v, irow_v,
             ubuf0, ubuf1, ibuf0, ibuf1, out_v,
             usem0, usem1, isem0, isem1):
    wid = lax.axis_index("s") * NC + lax.axis_index("c")

    pltpu.sync_copy(uidx_hbm.at[wid], uidx_v)
    pltpu.sync_copy(iidx_hbm.at[wid], iidx_v)

    for k in range(BPW // L):
        sl = pl.ds(k * L, L)
        urow_v[sl] = uidx_v[sl] >> 2
        irow_v[sl] = iidx_v[sl] >> 2

    ubufs = (ubuf0, ubuf1)
    ibufs = (ibuf0, ibuf1)
    usems = (usem0, usem1)
    isems = (isem0, isem1)

    def fire(c):
        sl = pl.ds(c * CHUNK, CHUNK)
        return (
            pltpu.async_copy(utab_hbm.at[urow_v.at[sl]], ubufs[c % 2],
                             usems[c % 2]),
            pltpu.async_copy(itab_hbm.at[irow_v.at[sl]], ibufs[c % 2],
                             isems[c % 2]),
        )

    lane_iota = lax.iota(jnp.int32, L)

    def compute(c):
        ub, ib = ubufs[c % 2], ibufs[c % 2]

        def blk_body(kb, carry):
            rows = kb * L + lane_iota
            sl = pl.ds(c * CHUNK + kb * L, L)
            uoff = (uidx_v[sl] & (PACK - 1)) << 5
            ioff = (iidx_v[sl] & (PACK - 1)) << 5
            acc = jnp.zeros((L,), jnp.float32)
            for j in range(D):
                u = plsc.load_gather(ub, [rows, uoff + j])
                v = plsc.load_gather(ib, [rows, ioff + j])
                acc = acc + u * v
            out_v[sl] = 1.0 / (1.0 + jnp.exp(-acc))
            return carry

        lax.fori_loop(0, BLKS, blk_body, 0)

    cps = fire(0)
    for c in range(NCHUNK):
        nxt = fire(c + 1) if c + 1 < NCHUNK else None
        for cp in cps:
            cp.wait()
        compute(c)
        cps = nxt

    pltpu.sync_copy(out_v, out_hbm.at[pl.ds(wid * BPW, BPW)])


@jax.jit
def _run(uidx, iidx, utab4, itab4):
    mesh = plsc.VectorSubcoreMesh(core_axis_name="c", subcore_axis_name="s")
    return pl.kernel(
        _sc_body,
        out_type=jax.ShapeDtypeStruct((B,), jnp.float32),
        mesh=mesh,
        scratch_types=[
            pltpu.VMEM((BPW,), jnp.int32),
            pltpu.VMEM((BPW,), jnp.int32),
            pltpu.VMEM((BPW,), jnp.int32),
            pltpu.VMEM((BPW,), jnp.int32),
            pltpu.VMEM((CHUNK, 128), jnp.float32),
            pltpu.VMEM((CHUNK, 128), jnp.float32),
            pltpu.VMEM((CHUNK, 128), jnp.float32),
            pltpu.VMEM((CHUNK, 128), jnp.float32),
            pltpu.VMEM((BPW,), jnp.float32),
            pltpu.SemaphoreType.DMA,
            pltpu.SemaphoreType.DMA,
            pltpu.SemaphoreType.DMA,
            pltpu.SemaphoreType.DMA,
        ],
        compiler_params=pltpu.CompilerParams(needs_layout_passes=False),
    )(uidx, iidx, utab4, itab4)


def kernel(user_indices, item_indices, user_table, tgt_item_table):
    uidx = user_indices.astype(jnp.int32).reshape(NW, BPW)
    iidx = item_indices.astype(jnp.int32).reshape(NW, BPW)
    utab4 = user_table.reshape(user_table.shape[0] // PACK, 128)
    itab4 = tgt_item_table.reshape(tgt_item_table.shape[0] // PACK, 128)
    return _run(uidx, iidx, utab4, itab4)
